# baseline (speedup 1.0000x reference)
import functools

import jax
import jax.numpy as jnp
from jax import lax
from jax.experimental import pallas as pl
from jax.experimental.pallas import tpu as pltpu
from jax.experimental.pallas import tpu_sc as plsc

N = 10000
D_IN = 128
HID = 16
NUM_CLASSES = 16

NPAD = 10240
R = 1280
G = NPAD // R
NC = 2
NS = 16
NW = NC * NS
CH = 128
SL = 16
RPT = NPAD // NS
CORE_SHARE = (48, 112)

_mesh = plsc.VectorSubcoreMesh(core_axis_name="c", subcore_axis_name="s")


def _sc_degree(col2d, ones_v, zeros_n, kchunks):
  k0 = kchunks * CORE_SHARE[0] // 80
  k1 = kchunks * CORE_SHARE[1] // 80

  @functools.partial(
      pl.kernel,
      out_type=(jax.ShapeDtypeStruct((NPAD,), jnp.float32),
                jax.ShapeDtypeStruct((NPAD,), jnp.float32)),
      mesh=_mesh,
      compiler_params=pltpu.CompilerParams(use_tc_tiling_on_sc=False),
      scratch_types=[
          pltpu.VMEM((max(k0, k1) * CH,), jnp.int32),
          pltpu.VMEM((max(k0, k1) * CH,), jnp.float32),
          pltpu.VMEM_SHARED((NPAD,), jnp.float32),
          pltpu.SemaphoreType.DMA,
      ],
  )
  def k(col_hbm, ones_hbm, zeros_hbm, out0, out1, colv, onesv, acc, sem):
    c = lax.axis_index("c")
    s = lax.axis_index("s")
    pltpu.sync_copy(zeros_hbm.at[pl.ds(s * RPT, RPT)],
                    acc.at[pl.ds(s * RPT, RPT)])

    def run(base_ch, kc):
      ec = kc * CH
      pltpu.sync_copy(col_hbm.at[pl.ds(base_ch * CH, ec)],
                      colv.at[pl.ds(0, ec)])
      pltpu.sync_copy(ones_hbm.at[pl.ds(0, ec)], onesv.at[pl.ds(0, ec)])
      plsc.subcore_barrier()
      pltpu.sync_copy(onesv.at[pl.ds(0, ec)],
                      acc.at[colv.at[pl.ds(0, ec)]], add=True)
      plsc.subcore_barrier()

    @pl.when(c == 0)
    def _():
      run(s * k0, k0)

    @pl.when(c == 1)
    def _():
      run(NS * k0 + s * k1, k1)

    @pl.when(c == 0)
    def _():
      pltpu.sync_copy(acc.at[pl.ds(s * RPT, RPT)], out0.at[pl.ds(s * RPT, RPT)])

    @pl.when(c == 1)
    def _():
      pltpu.sync_copy(acc.at[pl.ds(s * RPT, RPT)], out1.at[pl.ds(s * RPT, RPT)])

  return k(col2d, ones_v, zeros_n)


def _sc_scatter(hs, row2d, col2d, zeros_nh, kchunks):
  k0 = kchunks * CORE_SHARE[0] // 80
  k1 = kchunks * CORE_SHARE[1] // 80

  @functools.partial(
      pl.kernel,
      out_type=(jax.ShapeDtypeStruct((NPAD, HID), jnp.float32),
                jax.ShapeDtypeStruct((NPAD, HID), jnp.float32)),
      mesh=_mesh,
      compiler_params=pltpu.CompilerParams(use_tc_tiling_on_sc=False),
      scratch_types=[
          pltpu.VMEM((max(k0, k1) * CH,), jnp.int32),
          pltpu.VMEM((max(k0, k1) * CH,), jnp.int32),
          [pltpu.VMEM((SL * CH, HID), jnp.float32)] * 2,
          pltpu.VMEM_SHARED((NPAD, HID), jnp.float32),
          [pltpu.SemaphoreType.DMA] * 2,
      ],
  )
  def k(hs_hbm, row_hbm, col_hbm, zeros_hbm, out0, out1,
        rowv, colv, bufs, acc, gsems):
    c = lax.axis_index("c")
    s = lax.axis_index("s")
    sle = SL * CH
    pltpu.sync_copy(zeros_hbm.at[pl.ds(s * RPT, RPT)],
                    acc.at[pl.ds(s * RPT, RPT)])

    def run(base_ch, kc):
      ec = kc * CH
      pltpu.sync_copy(row_hbm.at[pl.ds(base_ch * CH, ec)],
                      rowv.at[pl.ds(0, ec)])
      pltpu.sync_copy(col_hbm.at[pl.ds(base_ch * CH, ec)],
                      colv.at[pl.ds(0, ec)])
      plsc.subcore_barrier()
      nsl = kc // SL
      pltpu.async_copy(hs_hbm.at[rowv.at[pl.ds(0, sle)]], bufs[0], gsems[0])
      for i in range(nsl):
        b = i % 2
        if i + 1 < nsl:
          pltpu.async_copy(hs_hbm.at[rowv.at[pl.ds((i + 1) * sle, sle)]],
                           bufs[1 - b], gsems[1 - b])
        pltpu.make_async_copy(hs_hbm.at[rowv.at[pl.ds(i * sle, sle)]],
                              bufs[b], gsems[b]).wait()
        pltpu.sync_copy(bufs[b], acc.at[colv.at[pl.ds(i * sle, sle)]],
                        add=True)
      plsc.subcore_barrier()

    @pl.when(c == 0)
    def _():
      run(s * k0, k0)

    @pl.when(c == 1)
    def _():
      run(NS * k0 + s * k1, k1)

    @pl.when(c == 0)
    def _():
      pltpu.sync_copy(acc.at[pl.ds(s * RPT, RPT)], out0.at[pl.ds(s * RPT, RPT)])

    @pl.when(c == 1)
    def _():
      pltpu.sync_copy(acc.at[pl.ds(s * RPT, RPT)], out1.at[pl.ds(s * RPT, RPT)])

  return k(hs, row2d, col2d, zeros_nh)


def _rowspec(cols):
  return pl.BlockSpec((R, cols), lambda i: (i, 0))


def _full(shape):
  return pl.BlockSpec(shape, lambda i: tuple(0 for _ in shape))


def _tc_conv1(x_pad, W1, dega, degb):
  def body(x_ref, w_ref, da, db, dis_ref, hs_ref):
    xw = jnp.dot(x_ref[...], w_ref[...], preferred_element_type=jnp.float32)
    deg = da[...] + db[...] + 1.0
    dis = lax.rsqrt(deg)
    dis_ref[...] = jnp.broadcast_to(dis, (R, HID))
    hs_ref[...] = xw * dis

  return pl.pallas_call(
      body, grid=(G,),
      in_specs=[_rowspec(D_IN), _full((D_IN, HID)), _rowspec(1), _rowspec(1)],
      out_specs=[_rowspec(HID), _rowspec(HID)],
      out_shape=[jax.ShapeDtypeStruct((NPAD, HID), jnp.float32),
                 jax.ShapeDtypeStruct((NPAD, HID), jnp.float32)],
  )(x_pad, W1, dega, degb)


def _tc_conv2(s1a, s1b, hs1, dis, b1, W2):
  def body(sa, sb, hs_ref, dis_ref, b_ref, w_ref, x1_ref, hs2_ref):
    d = dis_ref[...]
    x1 = jax.nn.relu(d * (sa[...] + sb[...] + hs_ref[...]) + b_ref[0:1, :])
    x1_ref[...] = x1
    hs2_ref[...] = d * jnp.dot(x1, w_ref[...],
                               preferred_element_type=jnp.float32)

  return pl.pallas_call(
      body, grid=(G,),
      in_specs=[_rowspec(HID)] * 4 + [_full((8, HID)), _full((HID, HID))],
      out_specs=[_rowspec(HID), _rowspec(HID)],
      out_shape=[jax.ShapeDtypeStruct((NPAD, HID), jnp.float32),
                 jax.ShapeDtypeStruct((NPAD, HID), jnp.float32)],
  )(s1a, s1b, hs1, dis, b1, W2)


def _tc_jk(s2a, s2b, hs2, dis, b2, x1,
           WihfT, WhhfT, bf, WihbT, WhhbT, bb, WattT):
  def lstm_step(x, h, c, WiT, WhT, b):
    gates = jnp.dot(x, WiT, preferred_element_type=jnp.float32) + b
    if h is not None:
      gates = gates + jnp.dot(h, WhT, preferred_element_type=jnp.float32)
    i, f, g, o = jnp.split(gates, 4, axis=-1)
    cn = jax.nn.sigmoid(i) * jnp.tanh(g)
    if c is not None:
      cn = cn + jax.nn.sigmoid(f) * c
    hn = jax.nn.sigmoid(o) * jnp.tanh(cn)
    return hn, cn

  def body(sa, sb, hs_ref, dis_ref, b2_ref, x1_ref,
           wifT, whfT, bf_ref, wibT, whbT, bb_ref, wattT, hs3_ref):
    d = dis_ref[...]
    x1 = x1_ref[...]
    x2 = jax.nn.relu(d * (sa[...] + sb[...] + hs_ref[...]) + b2_ref[0:1, :])
    bfv = bf_ref[0:1, :]
    bbv = bb_ref[0:1, :]
    hf1, cf1 = lstm_step(x1, None, None, wifT[...], None, bfv)
    hf2, _ = lstm_step(x2, hf1, cf1, wifT[...], whfT[...], bfv)
    hb2, cb2 = lstm_step(x2, None, None, wibT[...], None, bbv)
    hb1, _ = lstm_step(x1, hb2, cb2, wibT[...], whbT[...], bbv)
    w = wattT[...]
    s0 = jnp.dot(jnp.concatenate([hf1, hb1], 1), w,
                 preferred_element_type=jnp.float32)
    s1 = jnp.dot(jnp.concatenate([hf2, hb2], 1), w,
                 preferred_element_type=jnp.float32)
    m = jnp.maximum(s0, s1)
    e0 = jnp.exp(s0 - m)
    e1 = jnp.exp(s1 - m)
    xjk = (e0 * x1 + e1 * x2) / (e0 + e1)
    hs3_ref[...] = d * xjk

  return pl.pallas_call(
      body, grid=(G,),
      in_specs=[_rowspec(HID)] * 4 + [_full((8, HID)), _rowspec(HID),
                _full((HID, 128)), _full((2 * HID, 128)), _full((8, 128)),
                _full((HID, 128)), _full((2 * HID, 128)), _full((8, 128)),
                _full((4 * HID, HID))],
      out_specs=_rowspec(HID),
      out_shape=jax.ShapeDtypeStruct((NPAD, HID), jnp.float32),
  )(s2a, s2b, hs2, dis, b2, x1, WihfT, WhhfT, bf, WihbT, WhhbT, bb, WattT)


def _tc_final(s3a, s3b, hs3, dis, Wlin, blin):
  def body(sa, sb, hs_ref, dis_ref, w_ref, b_ref, o_ref):
    xprop = dis_ref[...] * (sa[...] + sb[...] + hs_ref[...])
    z = jnp.dot(xprop, w_ref[...],
                preferred_element_type=jnp.float32) + b_ref[0:1, :]
    m = jnp.max(z, axis=1, keepdims=True)
    ez = jnp.exp(z - m)
    o_ref[...] = z - m - jnp.log(jnp.sum(ez, axis=1, keepdims=True))

  return pl.pallas_call(
      body, grid=(G,),
      in_specs=[_rowspec(HID)] * 4 + [_full((HID, NUM_CLASSES)),
                _full((8, NUM_CLASSES))],
      out_specs=_rowspec(NUM_CLASSES),
      out_shape=jax.ShapeDtypeStruct((NPAD, NUM_CLASSES), jnp.float32),
  )(s3a, s3b, hs3, dis, Wlin, blin)


def kernel(x, edge_index, W1, b1, W2, b2, Wih_f, Whh_f, bih_f, bhh_f,
           Wih_b, Whh_b, bih_b, bhh_b, Watt, batt, Wlin, blin):
  E = edge_index.shape[1]
  kchunks = -(-E // (NW * CH * 8)) * 8
  EP = kchunks * NW * CH

  row = edge_index[0].astype(jnp.int32)
  col = edge_index[1].astype(jnp.int32)
  pad = jnp.full((EP - E,), N, jnp.int32)
  row1d = jnp.concatenate([row, pad])
  col1d = jnp.concatenate([col, pad])

  x_pad = jnp.zeros((NPAD, D_IN), jnp.float32).at[:N].set(x)
  zeros_n = jnp.zeros((NPAD,), jnp.float32)
  zeros_nh = jnp.zeros((NPAD, HID), jnp.float32)
  ones_v = None

  b1b = jnp.broadcast_to(b1[None, :], (8, HID))
  b2b = jnp.broadcast_to(b2[None, :], (8, HID))
  bfb = jnp.broadcast_to((bih_f + bhh_f)[None, :], (8, 128))
  bbb = jnp.broadcast_to((bih_b + bhh_b)[None, :], (8, 128))
  WattT = jnp.broadcast_to(Watt.T, (4 * HID, HID))
  blinb = jnp.broadcast_to(blin[None, :], (8, NUM_CLASSES))

  ones1d = jnp.ones((kchunks * max(CORE_SHARE) // 80 * CH,), jnp.float32)
  dega, degb = _sc_degree(col1d, ones1d, zeros_n, kchunks)
  dis, hs1 = _tc_conv1(x_pad, W1, dega[:, None], degb[:, None])

  s1a, s1b = _sc_scatter(hs1, row1d, col1d, zeros_nh, kchunks)
  x1, hs2 = _tc_conv2(s1a, s1b, hs1, dis, b1b, W2)

  s2a, s2b = _sc_scatter(hs2, row1d, col1d, zeros_nh, kchunks)
  hs3 = _tc_jk(s2a, s2b, hs2, dis, b2b, x1,
               Wih_f.T, Whh_f.T, bfb, Wih_b.T, Whh_b.T, bbb, WattT)

  s3a, s3b = _sc_scatter(hs3, row1d, col1d, zeros_nh, kchunks)
  out = _tc_final(s3a, s3b, hs3, dis, Wlin, blinb)
  return out[:N]


# speedup vs baseline: 1.0221x; 1.0221x over previous
import functools

import jax
import jax.numpy as jnp
from jax import lax
from jax.experimental import pallas as pl
from jax.experimental.pallas import tpu as pltpu
from jax.experimental.pallas import tpu_sc as plsc

N = 10000
D_IN = 128
HID = 16
NUM_CLASSES = 16

NPAD = 10240
R = 1280
G = NPAD // R
NC = 2
NS = 16
NW = NC * NS
CH = 128
SL = 20
RPT = NPAD // NS

_mesh = plsc.VectorSubcoreMesh(core_axis_name="c", subcore_axis_name="s")


def _sc_degree(col2d, ones_v, zeros_n, kchunks):
  @functools.partial(
      pl.kernel,
      out_type=(jax.ShapeDtypeStruct((NPAD,), jnp.float32),
                jax.ShapeDtypeStruct((NPAD,), jnp.float32)),
      mesh=_mesh,
      compiler_params=pltpu.CompilerParams(use_tc_tiling_on_sc=False),
      scratch_types=[
          pltpu.VMEM((kchunks * CH,), jnp.int32),
          pltpu.VMEM((kchunks * CH,), jnp.float32),
          pltpu.VMEM_SHARED((NPAD,), jnp.float32),
          pltpu.SemaphoreType.DMA,
      ],
  )
  def k(col_hbm, ones_hbm, zeros_hbm, out0, out1, colv, onesv, acc, sem):
    c = lax.axis_index("c")
    s = lax.axis_index("s")
    tid = c * NS + s
    ec = kchunks * CH
    pltpu.sync_copy(zeros_hbm.at[pl.ds(s * RPT, RPT)],
                    acc.at[pl.ds(s * RPT, RPT)])
    pltpu.sync_copy(col_hbm.at[pl.ds(tid * ec, ec)], colv)
    pltpu.sync_copy(ones_hbm, onesv)
    plsc.subcore_barrier()

    pltpu.sync_copy(onesv, acc.at[colv], add=True)
    plsc.subcore_barrier()

    @pl.when(c == 0)
    def _():
      pltpu.sync_copy(acc.at[pl.ds(s * RPT, RPT)], out0.at[pl.ds(s * RPT, RPT)])

    @pl.when(c == 1)
    def _():
      pltpu.sync_copy(acc.at[pl.ds(s * RPT, RPT)], out1.at[pl.ds(s * RPT, RPT)])

  return k(col2d, ones_v, zeros_n)


def _sc_scatter(hs, row2d, col2d, zeros_nh, kchunks):
  @functools.partial(
      pl.kernel,
      out_type=(jax.ShapeDtypeStruct((NPAD, HID), jnp.float32),
                jax.ShapeDtypeStruct((NPAD, HID), jnp.float32)),
      mesh=_mesh,
      compiler_params=pltpu.CompilerParams(use_tc_tiling_on_sc=False),
      scratch_types=[
          pltpu.VMEM((kchunks * CH,), jnp.int32),
          pltpu.VMEM((kchunks * CH,), jnp.int32),
          [pltpu.VMEM((SL * CH, HID), jnp.float32)] * 2,
          pltpu.VMEM_SHARED((NPAD, HID), jnp.float32),
          [pltpu.SemaphoreType.DMA] * 2,
      ],
  )
  def k(hs_hbm, row_hbm, col_hbm, zeros_hbm, out0, out1,
        rowv, colv, bufs, acc, gsems):
    c = lax.axis_index("c")
    s = lax.axis_index("s")
    tid = c * NS + s
    ec = kchunks * CH
    sle = SL * CH
    pltpu.sync_copy(zeros_hbm.at[pl.ds(s * RPT, RPT)],
                    acc.at[pl.ds(s * RPT, RPT)])
    pltpu.sync_copy(row_hbm.at[pl.ds(tid * ec, ec)], rowv)
    pltpu.sync_copy(col_hbm.at[pl.ds(tid * ec, ec)], colv)
    plsc.subcore_barrier()

    nsl = kchunks // SL
    pltpu.async_copy(hs_hbm.at[rowv.at[pl.ds(0, sle)]], bufs[0], gsems[0])
    for i in range(nsl):
      b = i % 2
      if i + 1 < nsl:
        pltpu.async_copy(hs_hbm.at[rowv.at[pl.ds((i + 1) * sle, sle)]],
                         bufs[1 - b], gsems[1 - b])
      pltpu.make_async_copy(hs_hbm.at[rowv.at[pl.ds(i * sle, sle)]],
                            bufs[b], gsems[b]).wait()
      pltpu.sync_copy(bufs[b], acc.at[colv.at[pl.ds(i * sle, sle)]], add=True)
    plsc.subcore_barrier()

    @pl.when(c == 0)
    def _():
      pltpu.sync_copy(acc.at[pl.ds(s * RPT, RPT)], out0.at[pl.ds(s * RPT, RPT)])

    @pl.when(c == 1)
    def _():
      pltpu.sync_copy(acc.at[pl.ds(s * RPT, RPT)], out1.at[pl.ds(s * RPT, RPT)])

  return k(hs, row2d, col2d, zeros_nh)


def _rowspec(cols):
  return pl.BlockSpec((R, cols), lambda i: (i, 0))


def _full(shape):
  return pl.BlockSpec(shape, lambda i: tuple(0 for _ in shape))


def _tc_conv1(x_pad, W1, dega, degb):
  def body(x_ref, w_ref, da, db, dis_ref, hs_ref):
    xw = jnp.dot(x_ref[...], w_ref[...], preferred_element_type=jnp.float32)
    deg = da[...] + db[...] + 1.0
    dis = lax.rsqrt(deg)
    dis_ref[...] = jnp.broadcast_to(dis, (R, HID))
    hs_ref[...] = xw * dis

  return pl.pallas_call(
      body, grid=(G,),
      in_specs=[_rowspec(D_IN), _full((D_IN, HID)), _rowspec(1), _rowspec(1)],
      out_specs=[_rowspec(HID), _rowspec(HID)],
      out_shape=[jax.ShapeDtypeStruct((NPAD, HID), jnp.float32),
                 jax.ShapeDtypeStruct((NPAD, HID), jnp.float32)],
  )(x_pad, W1, dega, degb)


def _tc_conv2(s1a, s1b, hs1, dis, b1, W2):
  def body(sa, sb, hs_ref, dis_ref, b_ref, w_ref, x1_ref, hs2_ref):
    d = dis_ref[...]
    x1 = jax.nn.relu(d * (sa[...] + sb[...] + hs_ref[...]) + b_ref[0:1, :])
    x1_ref[...] = x1
    hs2_ref[...] = d * jnp.dot(x1, w_ref[...],
                               preferred_element_type=jnp.float32)

  return pl.pallas_call(
      body, grid=(G,),
      in_specs=[_rowspec(HID)] * 4 + [_full((8, HID)), _full((HID, HID))],
      out_specs=[_rowspec(HID), _rowspec(HID)],
      out_shape=[jax.ShapeDtypeStruct((NPAD, HID), jnp.float32),
                 jax.ShapeDtypeStruct((NPAD, HID), jnp.float32)],
  )(s1a, s1b, hs1, dis, b1, W2)


def _tc_jk(s2a, s2b, hs2, dis, b2, x1,
           WihfT, WhhfT, bf, WihbT, WhhbT, bb, WattT):
  def lstm_step(x, h, c, WiT, WhT, b):
    gates = jnp.dot(x, WiT, preferred_element_type=jnp.float32) + b
    if h is not None:
      gates = gates + jnp.dot(h, WhT, preferred_element_type=jnp.float32)
    i, f, g, o = jnp.split(gates, 4, axis=-1)
    cn = jax.nn.sigmoid(i) * jnp.tanh(g)
    if c is not None:
      cn = cn + jax.nn.sigmoid(f) * c
    hn = jax.nn.sigmoid(o) * jnp.tanh(cn)
    return hn, cn

  def body(sa, sb, hs_ref, dis_ref, b2_ref, x1_ref,
           wifT, whfT, bf_ref, wibT, whbT, bb_ref, wattT, hs3_ref):
    d = dis_ref[...]
    x1 = x1_ref[...]
    x2 = jax.nn.relu(d * (sa[...] + sb[...] + hs_ref[...]) + b2_ref[0:1, :])
    bfv = bf_ref[0:1, :]
    bbv = bb_ref[0:1, :]
    hf1, cf1 = lstm_step(x1, None, None, wifT[...], None, bfv)
    hf2, _ = lstm_step(x2, hf1, cf1, wifT[...], whfT[...], bfv)
    hb2, cb2 = lstm_step(x2, None, None, wibT[...], None, bbv)
    hb1, _ = lstm_step(x1, hb2, cb2, wibT[...], whbT[...], bbv)
    w = wattT[...]
    s0 = jnp.dot(jnp.concatenate([hf1, hb1], 1), w,
                 preferred_element_type=jnp.float32)
    s1 = jnp.dot(jnp.concatenate([hf2, hb2], 1), w,
                 preferred_element_type=jnp.float32)
    m = jnp.maximum(s0, s1)
    e0 = jnp.exp(s0 - m)
    e1 = jnp.exp(s1 - m)
    xjk = (e0 * x1 + e1 * x2) / (e0 + e1)
    hs3_ref[...] = d * xjk

  return pl.pallas_call(
      body, grid=(G,),
      in_specs=[_rowspec(HID)] * 4 + [_full((8, HID)), _rowspec(HID),
                _full((HID, 128)), _full((2 * HID, 128)), _full((8, 128)),
                _full((HID, 128)), _full((2 * HID, 128)), _full((8, 128)),
                _full((4 * HID, HID))],
      out_specs=_rowspec(HID),
      out_shape=jax.ShapeDtypeStruct((NPAD, HID), jnp.float32),
  )(s2a, s2b, hs2, dis, b2, x1, WihfT, WhhfT, bf, WihbT, WhhbT, bb, WattT)


def _tc_final(s3a, s3b, hs3, dis, Wlin, blin):
  def body(sa, sb, hs_ref, dis_ref, w_ref, b_ref, o_ref):
    xprop = dis_ref[...] * (sa[...] + sb[...] + hs_ref[...])
    z = jnp.dot(xprop, w_ref[...],
                preferred_element_type=jnp.float32) + b_ref[0:1, :]
    m = jnp.max(z, axis=1, keepdims=True)
    ez = jnp.exp(z - m)
    o_ref[...] = z - m - jnp.log(jnp.sum(ez, axis=1, keepdims=True))

  return pl.pallas_call(
      body, grid=(G,),
      in_specs=[_rowspec(HID)] * 4 + [_full((HID, NUM_CLASSES)),
                _full((8, NUM_CLASSES))],
      out_specs=_rowspec(NUM_CLASSES),
      out_shape=jax.ShapeDtypeStruct((NPAD, NUM_CLASSES), jnp.float32),
  )(s3a, s3b, hs3, dis, Wlin, blin)


def kernel(x, edge_index, W1, b1, W2, b2, Wih_f, Whh_f, bih_f, bhh_f,
           Wih_b, Whh_b, bih_b, bhh_b, Watt, batt, Wlin, blin):
  E = edge_index.shape[1]
  kchunks = -(-E // (NW * CH * 8)) * 8
  EP = kchunks * NW * CH

  row = edge_index[0].astype(jnp.int32)
  col = edge_index[1].astype(jnp.int32)
  pad = jnp.full((EP - E,), N, jnp.int32)
  row1d = jnp.concatenate([row, pad])
  col1d = jnp.concatenate([col, pad])

  x_pad = jnp.zeros((NPAD, D_IN), jnp.float32).at[:N].set(x)
  zeros_n = jnp.zeros((NPAD,), jnp.float32)
  zeros_nh = jnp.zeros((NPAD, HID), jnp.float32)
  ones_v = None

  b1b = jnp.broadcast_to(b1[None, :], (8, HID))
  b2b = jnp.broadcast_to(b2[None, :], (8, HID))
  bfb = jnp.broadcast_to((bih_f + bhh_f)[None, :], (8, 128))
  bbb = jnp.broadcast_to((bih_b + bhh_b)[None, :], (8, 128))
  WattT = jnp.broadcast_to(Watt.T, (4 * HID, HID))
  blinb = jnp.broadcast_to(blin[None, :], (8, NUM_CLASSES))

  ones1d = jnp.ones((kchunks * CH,), jnp.float32)
  dega, degb = _sc_degree(col1d, ones1d, zeros_n, kchunks)
  dis, hs1 = _tc_conv1(x_pad, W1, dega[:, None], degb[:, None])

  s1a, s1b = _sc_scatter(hs1, row1d, col1d, zeros_nh, kchunks)
  x1, hs2 = _tc_conv2(s1a, s1b, hs1, dis, b1b, W2)

  s2a, s2b = _sc_scatter(hs2, row1d, col1d, zeros_nh, kchunks)
  hs3 = _tc_jk(s2a, s2b, hs2, dis, b2b, x1,
               Wih_f.T, Whh_f.T, bfb, Wih_b.T, Whh_b.T, bbb, WattT)

  s3a, s3b = _sc_scatter(hs3, row1d, col1d, zeros_nh, kchunks)
  out = _tc_final(s3a, s3b, hs3, dis, Wlin, blinb)
  return out[:N]


# speedup vs baseline: 1.0391x; 1.0167x over previous
import functools

import jax
import jax.numpy as jnp
from jax import lax
from jax.experimental import pallas as pl
from jax.experimental.pallas import tpu as pltpu
from jax.experimental.pallas import tpu_sc as plsc

N = 10000
D_IN = 128
HID = 16
NUM_CLASSES = 16

NPAD = 10240
R = 1280
G = NPAD // R
NC = 2
NS = 16
NW = NC * NS
CH = 128
SL = 8
RPT = NPAD // NS

_mesh = plsc.VectorSubcoreMesh(core_axis_name="c", subcore_axis_name="s")


def _sc_degree(col2d, ones_v, zeros_n, kchunks):
  @functools.partial(
      pl.kernel,
      out_type=(jax.ShapeDtypeStruct((NPAD,), jnp.float32),
                jax.ShapeDtypeStruct((NPAD,), jnp.float32)),
      mesh=_mesh,
      compiler_params=pltpu.CompilerParams(use_tc_tiling_on_sc=False),
      scratch_types=[
          pltpu.VMEM((kchunks * CH,), jnp.int32),
          pltpu.VMEM((kchunks * CH,), jnp.float32),
          pltpu.VMEM_SHARED((NPAD,), jnp.float32),
          pltpu.SemaphoreType.DMA,
      ],
  )
  def k(col_hbm, ones_hbm, zeros_hbm, out0, out1, colv, onesv, acc, sem):
    c = lax.axis_index("c")
    s = lax.axis_index("s")
    tid = c * NS + s
    ec = kchunks * CH
    pltpu.sync_copy(zeros_hbm.at[pl.ds(s * RPT, RPT)],
                    acc.at[pl.ds(s * RPT, RPT)])
    pltpu.sync_copy(col_hbm.at[pl.ds(tid * ec, ec)], colv)
    pltpu.sync_copy(ones_hbm, onesv)
    plsc.subcore_barrier()

    pltpu.sync_copy(onesv, acc.at[colv], add=True)
    plsc.subcore_barrier()

    @pl.when(c == 0)
    def _():
      pltpu.sync_copy(acc.at[pl.ds(s * RPT, RPT)], out0.at[pl.ds(s * RPT, RPT)])

    @pl.when(c == 1)
    def _():
      pltpu.sync_copy(acc.at[pl.ds(s * RPT, RPT)], out1.at[pl.ds(s * RPT, RPT)])

  return k(col2d, ones_v, zeros_n)


def _sc_scatter(hs, row2d, col2d, zeros_nh, kchunks):
  @functools.partial(
      pl.kernel,
      out_type=(jax.ShapeDtypeStruct((NPAD, HID), jnp.float32),
                jax.ShapeDtypeStruct((NPAD, HID), jnp.float32)),
      mesh=_mesh,
      compiler_params=pltpu.CompilerParams(use_tc_tiling_on_sc=False),
      scratch_types=[
          pltpu.VMEM((kchunks * CH,), jnp.int32),
          pltpu.VMEM((kchunks * CH,), jnp.int32),
          [pltpu.VMEM((SL * CH, HID), jnp.float32)] * 2,
          pltpu.VMEM_SHARED((NPAD, HID), jnp.float32),
          [pltpu.SemaphoreType.DMA] * 2,
      ],
  )
  def k(hs_hbm, row_hbm, col_hbm, zeros_hbm, out0, out1,
        rowv, colv, bufs, acc, gsems):
    c = lax.axis_index("c")
    s = lax.axis_index("s")
    tid = c * NS + s
    ec = kchunks * CH
    sle = SL * CH
    pltpu.sync_copy(zeros_hbm.at[pl.ds(s * RPT, RPT)],
                    acc.at[pl.ds(s * RPT, RPT)])
    pltpu.sync_copy(row_hbm.at[pl.ds(tid * ec, ec)], rowv)
    pltpu.sync_copy(col_hbm.at[pl.ds(tid * ec, ec)], colv)
    plsc.subcore_barrier()

    nsl = kchunks // SL
    pltpu.async_copy(hs_hbm.at[rowv.at[pl.ds(0, sle)]], bufs[0], gsems[0])
    for i in range(nsl):
      b = i % 2
      if i + 1 < nsl:
        pltpu.async_copy(hs_hbm.at[rowv.at[pl.ds((i + 1) * sle, sle)]],
                         bufs[1 - b], gsems[1 - b])
      pltpu.make_async_copy(hs_hbm.at[rowv.at[pl.ds(i * sle, sle)]],
                            bufs[b], gsems[b]).wait()
      pltpu.sync_copy(bufs[b], acc.at[colv.at[pl.ds(i * sle, sle)]], add=True)
    plsc.subcore_barrier()

    @pl.when(c == 0)
    def _():
      pltpu.sync_copy(acc.at[pl.ds(s * RPT, RPT)], out0.at[pl.ds(s * RPT, RPT)])

    @pl.when(c == 1)
    def _():
      pltpu.sync_copy(acc.at[pl.ds(s * RPT, RPT)], out1.at[pl.ds(s * RPT, RPT)])

  return k(hs, row2d, col2d, zeros_nh)


def _rowspec(cols):
  return pl.BlockSpec((R, cols), lambda i: (i, 0))


def _full(shape):
  return pl.BlockSpec(shape, lambda i: tuple(0 for _ in shape))


def _tc_conv1(x_pad, W1, dega, degb):
  def body(x_ref, w_ref, da, db, dis_ref, hs_ref):
    xw = jnp.dot(x_ref[...], w_ref[...], preferred_element_type=jnp.float32)
    deg = da[...] + db[...] + 1.0
    dis = lax.rsqrt(deg)
    dis_ref[...] = jnp.broadcast_to(dis, (R, HID))
    hs_ref[...] = xw * dis

  return pl.pallas_call(
      body, grid=(G,),
      in_specs=[_rowspec(D_IN), _full((D_IN, HID)), _rowspec(1), _rowspec(1)],
      out_specs=[_rowspec(HID), _rowspec(HID)],
      out_shape=[jax.ShapeDtypeStruct((NPAD, HID), jnp.float32),
                 jax.ShapeDtypeStruct((NPAD, HID), jnp.float32)],
  )(x_pad, W1, dega, degb)


def _tc_conv2(s1a, s1b, hs1, dis, b1, W2):
  def body(sa, sb, hs_ref, dis_ref, b_ref, w_ref, x1_ref, hs2_ref):
    d = dis_ref[...]
    x1 = jax.nn.relu(d * (sa[...] + sb[...] + hs_ref[...]) + b_ref[0:1, :])
    x1_ref[...] = x1
    hs2_ref[...] = d * jnp.dot(x1, w_ref[...],
                               preferred_element_type=jnp.float32)

  return pl.pallas_call(
      body, grid=(G,),
      in_specs=[_rowspec(HID)] * 4 + [_full((8, HID)), _full((HID, HID))],
      out_specs=[_rowspec(HID), _rowspec(HID)],
      out_shape=[jax.ShapeDtypeStruct((NPAD, HID), jnp.float32),
                 jax.ShapeDtypeStruct((NPAD, HID), jnp.float32)],
  )(s1a, s1b, hs1, dis, b1, W2)


def _tc_jk(s2a, s2b, hs2, dis, b2, x1,
           WihfT, WhhfT, bf, WihbT, WhhbT, bb, WattT):
  def lstm_step(x, h, c, WiT, WhT, b):
    gates = jnp.dot(x, WiT, preferred_element_type=jnp.float32) + b
    if h is not None:
      gates = gates + jnp.dot(h, WhT, preferred_element_type=jnp.float32)
    i, f, g, o = jnp.split(gates, 4, axis=-1)
    cn = jax.nn.sigmoid(i) * jnp.tanh(g)
    if c is not None:
      cn = cn + jax.nn.sigmoid(f) * c
    hn = jax.nn.sigmoid(o) * jnp.tanh(cn)
    return hn, cn

  def body(sa, sb, hs_ref, dis_ref, b2_ref, x1_ref,
           wifT, whfT, bf_ref, wibT, whbT, bb_ref, wattT, hs3_ref):
    d = dis_ref[...]
    x1 = x1_ref[...]
    x2 = jax.nn.relu(d * (sa[...] + sb[...] + hs_ref[...]) + b2_ref[0:1, :])
    bfv = bf_ref[0:1, :]
    bbv = bb_ref[0:1, :]
    hf1, cf1 = lstm_step(x1, None, None, wifT[...], None, bfv)
    hf2, _ = lstm_step(x2, hf1, cf1, wifT[...], whfT[...], bfv)
    hb2, cb2 = lstm_step(x2, None, None, wibT[...], None, bbv)
    hb1, _ = lstm_step(x1, hb2, cb2, wibT[...], whbT[...], bbv)
    w = wattT[...]
    s0 = jnp.dot(jnp.concatenate([hf1, hb1], 1), w,
                 preferred_element_type=jnp.float32)
    s1 = jnp.dot(jnp.concatenate([hf2, hb2], 1), w,
                 preferred_element_type=jnp.float32)
    m = jnp.maximum(s0, s1)
    e0 = jnp.exp(s0 - m)
    e1 = jnp.exp(s1 - m)
    xjk = (e0 * x1 + e1 * x2) / (e0 + e1)
    hs3_ref[...] = d * xjk

  return pl.pallas_call(
      body, grid=(G,),
      in_specs=[_rowspec(HID)] * 4 + [_full((8, HID)), _rowspec(HID),
                _full((HID, 128)), _full((2 * HID, 128)), _full((8, 128)),
                _full((HID, 128)), _full((2 * HID, 128)), _full((8, 128)),
                _full((4 * HID, HID))],
      out_specs=_rowspec(HID),
      out_shape=jax.ShapeDtypeStruct((NPAD, HID), jnp.float32),
  )(s2a, s2b, hs2, dis, b2, x1, WihfT, WhhfT, bf, WihbT, WhhbT, bb, WattT)


def _tc_final(s3a, s3b, hs3, dis, Wlin, blin):
  def body(sa, sb, hs_ref, dis_ref, w_ref, b_ref, o_ref):
    xprop = dis_ref[...] * (sa[...] + sb[...] + hs_ref[...])
    z = jnp.dot(xprop, w_ref[...],
                preferred_element_type=jnp.float32) + b_ref[0:1, :]
    m = jnp.max(z, axis=1, keepdims=True)
    ez = jnp.exp(z - m)
    o_ref[...] = z - m - jnp.log(jnp.sum(ez, axis=1, keepdims=True))

  return pl.pallas_call(
      body, grid=(G,),
      in_specs=[_rowspec(HID)] * 4 + [_full((HID, NUM_CLASSES)),
                _full((8, NUM_CLASSES))],
      out_specs=_rowspec(NUM_CLASSES),
      out_shape=jax.ShapeDtypeStruct((NPAD, NUM_CLASSES), jnp.float32),
  )(s3a, s3b, hs3, dis, Wlin, blin)


def kernel(x, edge_index, W1, b1, W2, b2, Wih_f, Whh_f, bih_f, bhh_f,
           Wih_b, Whh_b, bih_b, bhh_b, Watt, batt, Wlin, blin):
  E = edge_index.shape[1]
  kchunks = -(-E // (NW * CH * 8)) * 8
  EP = kchunks * NW * CH

  row = edge_index[0].astype(jnp.int32)
  col = edge_index[1].astype(jnp.int32)
  pad = jnp.full((EP - E,), N, jnp.int32)
  row1d = jnp.concatenate([row, pad])
  col1d = jnp.concatenate([col, pad])

  x_pad = jnp.zeros((NPAD, D_IN), jnp.float32).at[:N].set(x)
  zeros_n = jnp.zeros((NPAD,), jnp.float32)
  zeros_nh = jnp.zeros((NPAD, HID), jnp.float32)
  ones_v = None

  b1b = jnp.broadcast_to(b1[None, :], (8, HID))
  b2b = jnp.broadcast_to(b2[None, :], (8, HID))
  bfb = jnp.broadcast_to((bih_f + bhh_f)[None, :], (8, 128))
  bbb = jnp.broadcast_to((bih_b + bhh_b)[None, :], (8, 128))
  WattT = jnp.broadcast_to(Watt.T, (4 * HID, HID))
  blinb = jnp.broadcast_to(blin[None, :], (8, NUM_CLASSES))

  ones1d = jnp.ones((kchunks * CH,), jnp.float32)
  dega, degb = _sc_degree(col1d, ones1d, zeros_n, kchunks)
  dis, hs1 = _tc_conv1(x_pad, W1, dega[:, None], degb[:, None])

  s1a, s1b = _sc_scatter(hs1, row1d, col1d, zeros_nh, kchunks)
  x1, hs2 = _tc_conv2(s1a, s1b, hs1, dis, b1b, W2)

  s2a, s2b = _sc_scatter(hs2, row1d, col1d, zeros_nh, kchunks)
  hs3 = _tc_jk(s2a, s2b, hs2, dis, b2b, x1,
               Wih_f.T, Whh_f.T, bfb, Wih_b.T, Whh_b.T, bbb, WattT)

  s3a, s3b = _sc_scatter(hs3, row1d, col1d, zeros_nh, kchunks)
  out = _tc_final(s3a, s3b, hs3, dis, Wlin, blinb)
  return out[:N]
